# baseline (speedup 1.0000x reference)
import functools

import jax
import jax.numpy as jnp
from jax import lax
from jax.experimental import pallas as pl
from jax.experimental.pallas import tpu as pltpu
from jax.experimental.pallas import tpu_sc as plsc

N = 10000
E = 320000
D = 128

NW = 32
NPT = 313
NPAD = NW * NPT
C = 3200
CPV = C // 16
SCB = 8
NCHUNK = E // C
K = 256
CAP = C + K + 48
ACCW = (NPT + 1) * D
NEG = -3.0e38


def _sc_segment_max(nf_hbm, src_hbm, dst_hbm, out_hbm, dst_chunk0, dst_chunk1,
                    src_chunk0, src_chunk1, pair_list, proc_src, proc_dst,
                    rows, acc, sem_e0, sem_e1, sem_g0, sem_g1, sem_g2, sem_g3):
  wid = lax.axis_index("s") * 2 + lax.axis_index("c")
  lo = wid * NPT
  sem_e = (sem_e0, sem_e1)
  src_chunks = (src_chunk0, src_chunk1)
  dst_chunks = (dst_chunk0, dst_chunk1)

  def init_body(i, _):
    acc[pl.ds(i * 16, 16)] = jnp.full((16,), NEG, jnp.float32)
    return 0
  lax.fori_loop(0, ACCW // 16, init_body, 0)

  def drain(cnt):
    for w in range(K // 16):
      v = pair_list[pl.ds(w * 16, 16)]
      proc_src[w // 4, pl.ds((w % 4) * 16, 16)] = v >> 9
      proc_dst[pl.ds(w * 16, 16)] = v & 511

    sem_g = (sem_g0, sem_g1, sem_g2, sem_g3)
    cps = [pltpu.async_copy(nf_hbm.at[proc_src.at[q]],
                            rows.at[pl.ds(q * 64, 64)], sem_g[q])
           for q in range(K // 64)]

    def shift_body(j, _):
      pair_list[pl.ds(j * 16, 16)] = pair_list[pl.ds(K + j * 16, 16)]
      return 0
    lax.fori_loop(0, (cnt - K + 31) // 16, shift_body, 0)

    def acc_group(g, _):
      dlv = proc_dst[pl.ds(g * 16, 16)] * D
      e0 = g * 16
      r = [rows[e0, pl.ds(c * 16, 16)] for c in range(D // 16)]
      for j in range(16):
        base = dlv[j]
        a = [acc[pl.ds(base + c * 16, 16)] for c in range(D // 16)]
        m = [jnp.maximum(a[c], r[c]) for c in range(D // 16)]
        if j < 15:
          r = [rows[e0 + j + 1, pl.ds(c * 16, 16)] for c in range(D // 16)]
        for c in range(D // 16):
          acc[pl.ds(base + c * 16, 16)] = m[c]
      return 0
    for q in range(K // 64):
      cps[q].wait()
      lax.fori_loop(q * 4, (q + 1) * 4, acc_group, 0)
    return cnt - K

  def start_chunk(idx, b):
    pltpu.async_copy(src_hbm.at[pl.ds(idx * C, C)], src_chunks[b], sem_e[b])
    pltpu.async_copy(dst_hbm.at[pl.ds(idx * C, C)], dst_chunks[b], sem_e[b])

  def wait_chunk(b):
    for chunks in (src_chunks, dst_chunks):
      pltpu.make_async_copy(src_hbm.at[pl.ds(0, C)], chunks[b],
                            sem_e[b]).wait()

  def scan_chunk(b, cnt):
    def scan_body(i, cnt):
      vs = []
      pcs = []
      for u in range(SCB):
        d16 = dst_chunks[b][pl.ds(i * (16 * SCB) + u * 16, 16)]
        s16 = src_chunks[b][pl.ds(i * (16 * SCB) + u * 16, 16)]
        dl = d16 - lo
        mask = (dl >= 0) & (dl < NPT)
        packed = jnp.where(mask, (s16 << 9) + dl, jnp.int32(0x7FFFFFFF))
        vs.append(jnp.sort(packed))
        pcs.append(plsc.all_reduce_population_count(mask))
      for u in range(SCB):
        pair_list[pl.ds(cnt, 16)] = vs[u]
        cnt = cnt + pcs[u][0]
      return cnt
    return lax.fori_loop(0, CPV // SCB, scan_body, cnt)

  start_chunk(0, 0)

  def pair_body(p, cnt):
    wait_chunk(0)
    start_chunk(2 * p + 1, 1)
    cnt = scan_chunk(0, cnt)
    cnt = lax.while_loop(lambda c: c >= K, drain, cnt)

    wait_chunk(1)

    @pl.when(p < NCHUNK // 2 - 1)
    def _():
      start_chunk(2 * p + 2, 0)
    cnt = scan_chunk(1, cnt)
    return lax.while_loop(lambda c: c >= K, drain, cnt)

  cnt = lax.fori_loop(0, NCHUNK // 2, pair_body, jnp.int32(0))

  for w in range(K // 16):
    pair_list[pl.ds(cnt + w * 16, 16)] = jnp.full((16,), NPT, jnp.int32)
  drain(jnp.int32(K))

  pltpu.sync_copy(acc.at[pl.ds(0, NPT * D)], out_hbm.at[pl.ds(lo * D, NPT * D)])


@functools.partial(
    pl.kernel,
    mesh=plsc.VectorSubcoreMesh(core_axis_name="c", subcore_axis_name="s"),
    out_type=jax.ShapeDtypeStruct((NPAD * D,), jnp.float32),
    scratch_types=[
        pltpu.VMEM((C,), jnp.int32),
        pltpu.VMEM((C,), jnp.int32),
        pltpu.VMEM((C,), jnp.int32),
        pltpu.VMEM((C,), jnp.int32),
        pltpu.VMEM((CAP,), jnp.int32),
        pltpu.VMEM((K // 64, 64), jnp.int32),
        pltpu.VMEM((K,), jnp.int32),
        pltpu.VMEM((K, D), jnp.float32),
        pltpu.VMEM((ACCW,), jnp.float32),
        pltpu.SemaphoreType.DMA,
        pltpu.SemaphoreType.DMA,
        pltpu.SemaphoreType.DMA,
        pltpu.SemaphoreType.DMA,
        pltpu.SemaphoreType.DMA,
        pltpu.SemaphoreType.DMA,
    ],
    compiler_params=pltpu.CompilerParams(needs_layout_passes=False),
)
def _segment_max_kernel(nf_hbm, src_hbm, dst_hbm, out_hbm, *scratch):
  _sc_segment_max(nf_hbm, src_hbm, dst_hbm, out_hbm, *scratch)


RB = 2000


def _tc_mlp_body(x_ref, m_ref, w1_ref, w2_ref, b_ref, o_ref):
  x = x_ref[...]
  agg = m_ref[...] - x
  agg = jnp.where(agg < -10000.0, 0.0, agg)
  acc = jnp.dot(x, w1_ref[...], preferred_element_type=jnp.float32)
  acc += jnp.dot(agg, w2_ref[...], preferred_element_type=jnp.float32)
  o_ref[...] = jnp.maximum(acc + b_ref[...], 0.0)


def _tc_mlp(x, m, W, b):
  return pl.pallas_call(
      _tc_mlp_body,
      grid=(N // RB,),
      in_specs=[
          pl.BlockSpec((RB, D), lambda i: (i, 0)),
          pl.BlockSpec((RB, D), lambda i: (i, 0)),
          pl.BlockSpec((D, D), lambda i: (0, 0)),
          pl.BlockSpec((D, D), lambda i: (0, 0)),
          pl.BlockSpec((1, D), lambda i: (0, 0)),
      ],
      out_specs=pl.BlockSpec((RB, D), lambda i: (i, 0)),
      out_shape=jax.ShapeDtypeStruct((N, D), jnp.float32),
  )(x, m, W[:D], W[D:], b.reshape(1, D))


def kernel(n_feat, edge_index, W, b):
  m_flat = _segment_max_kernel(n_feat, edge_index[0], edge_index[1])
  m = m_flat.reshape(NPAD, D)[:N]
  return _tc_mlp(n_feat, m, W, b)


# speedup vs baseline: 1.0356x; 1.0356x over previous
import functools

import jax
import jax.numpy as jnp
from jax import lax
from jax.experimental import pallas as pl
from jax.experimental.pallas import tpu as pltpu
from jax.experimental.pallas import tpu_sc as plsc

N = 10000
E = 320000
D = 128

NW = 32
NPT = 313
NPAD = NW * NPT
C = 3200
CPV = C // 16
SCB = 8
NCHUNK = E // C
K = 256
CAP = C + K + 48
ACCW = (NPT + 1) * D
NEG = -3.0e38


def _sc_segment_max(nf_hbm, src_hbm, dst_hbm, out_hbm, dst_chunk0, dst_chunk1,
                    src_chunk0, src_chunk1, pair_list, proc_src, proc_dst,
                    rows, acc, sem_e0, sem_e1, sem_g0, sem_g1):
  wid = lax.axis_index("s") * 2 + lax.axis_index("c")
  lo = wid * NPT
  sem_e = (sem_e0, sem_e1)
  src_chunks = (src_chunk0, src_chunk1)
  dst_chunks = (dst_chunk0, dst_chunk1)

  def init_body(i, _):
    acc[pl.ds(i * 16, 16)] = jnp.full((16,), NEG, jnp.float32)
    return 0
  lax.fori_loop(0, ACCW // 16, init_body, 0)

  def drain(cnt):
    for w in range(K // 16):
      v = pair_list[pl.ds(w * 16, 16)]
      proc_src[w // 8, pl.ds((w % 8) * 16, 16)] = v >> 9
      proc_dst[pl.ds(w * 16, 16)] = v & 511

    sem_g = (sem_g0, sem_g1)
    cps = [pltpu.async_copy(nf_hbm.at[proc_src.at[q]],
                            rows.at[pl.ds(q * 128, 128)], sem_g[q])
           for q in range(K // 128)]

    def shift_body(j, _):
      pair_list[pl.ds(j * 16, 16)] = pair_list[pl.ds(K + j * 16, 16)]
      return 0
    lax.fori_loop(0, (cnt - K + 31) // 16, shift_body, 0)

    def acc_group(g, _):
      dlv = proc_dst[pl.ds(g * 16, 16)] * D
      e0 = g * 16
      r = [rows[e0, pl.ds(c * 16, 16)] for c in range(D // 16)]
      for j in range(16):
        base = dlv[j]
        a = [acc[pl.ds(base + c * 16, 16)] for c in range(D // 16)]
        m = [jnp.maximum(a[c], r[c]) for c in range(D // 16)]
        if j < 15:
          r = [rows[e0 + j + 1, pl.ds(c * 16, 16)] for c in range(D // 16)]
        for c in range(D // 16):
          acc[pl.ds(base + c * 16, 16)] = m[c]
      return 0
    for q in range(K // 128):
      cps[q].wait()
      lax.fori_loop(q * 8, (q + 1) * 8, acc_group, 0)
    return cnt - K

  def start_chunk(idx, b):
    pltpu.async_copy(src_hbm.at[pl.ds(idx * C, C)], src_chunks[b], sem_e[b])
    pltpu.async_copy(dst_hbm.at[pl.ds(idx * C, C)], dst_chunks[b], sem_e[b])

  def wait_chunk(b):
    for chunks in (src_chunks, dst_chunks):
      pltpu.make_async_copy(src_hbm.at[pl.ds(0, C)], chunks[b],
                            sem_e[b]).wait()

  def scan_chunk(b, cnt):
    def scan_body(i, cnt):
      vs = []
      pcs = []
      for u in range(SCB):
        d16 = dst_chunks[b][pl.ds(i * (16 * SCB) + u * 16, 16)]
        s16 = src_chunks[b][pl.ds(i * (16 * SCB) + u * 16, 16)]
        dl = d16 - lo
        mask = (dl >= 0) & (dl < NPT)
        packed = jnp.where(mask, (s16 << 9) + dl, jnp.int32(0x7FFFFFFF))
        vs.append(jnp.sort(packed))
        pcs.append(plsc.all_reduce_population_count(mask))
      for u in range(SCB):
        pair_list[pl.ds(cnt, 16)] = vs[u]
        cnt = cnt + pcs[u][0]
      return cnt
    return lax.fori_loop(0, CPV // SCB, scan_body, cnt)

  start_chunk(0, 0)

  def pair_body(p, cnt):
    wait_chunk(0)
    start_chunk(2 * p + 1, 1)
    cnt = scan_chunk(0, cnt)
    cnt = lax.while_loop(lambda c: c >= K, drain, cnt)

    wait_chunk(1)

    @pl.when(p < NCHUNK // 2 - 1)
    def _():
      start_chunk(2 * p + 2, 0)
    cnt = scan_chunk(1, cnt)
    return lax.while_loop(lambda c: c >= K, drain, cnt)

  cnt = lax.fori_loop(0, NCHUNK // 2, pair_body, jnp.int32(0))

  for w in range(K // 16):
    pair_list[pl.ds(cnt + w * 16, 16)] = jnp.full((16,), NPT, jnp.int32)
  drain(jnp.int32(K))

  pltpu.sync_copy(acc.at[pl.ds(0, NPT * D)], out_hbm.at[pl.ds(lo * D, NPT * D)])


@functools.partial(
    pl.kernel,
    mesh=plsc.VectorSubcoreMesh(core_axis_name="c", subcore_axis_name="s"),
    out_type=jax.ShapeDtypeStruct((NPAD * D,), jnp.float32),
    scratch_types=[
        pltpu.VMEM((C,), jnp.int32),
        pltpu.VMEM((C,), jnp.int32),
        pltpu.VMEM((C,), jnp.int32),
        pltpu.VMEM((C,), jnp.int32),
        pltpu.VMEM((CAP,), jnp.int32),
        pltpu.VMEM((K // 128, 128), jnp.int32),
        pltpu.VMEM((K,), jnp.int32),
        pltpu.VMEM((K, D), jnp.float32),
        pltpu.VMEM((ACCW,), jnp.float32),
        pltpu.SemaphoreType.DMA,
        pltpu.SemaphoreType.DMA,
        pltpu.SemaphoreType.DMA,
        pltpu.SemaphoreType.DMA,
    ],
    compiler_params=pltpu.CompilerParams(needs_layout_passes=False),
)
def _segment_max_kernel(nf_hbm, src_hbm, dst_hbm, out_hbm, *scratch):
  _sc_segment_max(nf_hbm, src_hbm, dst_hbm, out_hbm, *scratch)


RB = 2000


def _tc_mlp_body(x_ref, m_ref, w1_ref, w2_ref, b_ref, o_ref):
  x = x_ref[...]
  agg = m_ref[...] - x
  agg = jnp.where(agg < -10000.0, 0.0, agg)
  acc = jnp.dot(x, w1_ref[...], preferred_element_type=jnp.float32)
  acc += jnp.dot(agg, w2_ref[...], preferred_element_type=jnp.float32)
  o_ref[...] = jnp.maximum(acc + b_ref[...], 0.0)


def _tc_mlp(x, m, W, b):
  return pl.pallas_call(
      _tc_mlp_body,
      grid=(N // RB,),
      in_specs=[
          pl.BlockSpec((RB, D), lambda i: (i, 0)),
          pl.BlockSpec((RB, D), lambda i: (i, 0)),
          pl.BlockSpec((D, D), lambda i: (0, 0)),
          pl.BlockSpec((D, D), lambda i: (0, 0)),
          pl.BlockSpec((1, D), lambda i: (0, 0)),
      ],
      out_specs=pl.BlockSpec((RB, D), lambda i: (i, 0)),
      out_shape=jax.ShapeDtypeStruct((N, D), jnp.float32),
  )(x, m, W[:D], W[D:], b.reshape(1, D))


def kernel(n_feat, edge_index, W, b):
  m_flat = _segment_max_kernel(n_feat, edge_index[0], edge_index[1])
  m = m_flat.reshape(NPAD, D)[:N]
  return _tc_mlp(n_feat, m, W, b)
